# baseline (speedup 1.0000x reference)
import functools

import jax
import jax.numpy as jnp
from jax import lax
from jax.experimental import pallas as pl
from jax.experimental.pallas import tpu as pltpu
from jax.experimental.pallas import tpu_sc as plsc

_C = 8192


def _sc_body(B, F, rows_per_w, nchunks,
             white_hbm, black_hbm, turn_hbm, l0w_hbm, params_hbm, out_hbm,
             wf_buf, bf_buf, w0_buf, acc_buf, sums_buf, turn_buf, out_buf,
             params_buf, red_buf):
    nc = lax.axis_index("c")
    ns = lax.axis_index("s")
    wid = ns * 2 + nc
    base = wid * rows_per_w

    pltpu.sync_copy(params_hbm, params_buf)
    pltpu.sync_copy(turn_hbm.at[pl.ds(base, rows_per_w)], turn_buf)

    zero = jnp.zeros((16,), jnp.float32)

    def _zero_body(i, _):
        acc_buf[pl.ds(i * 16, 16)] = zero
        return 0

    lax.fori_loop(0, rows_per_w * 8, _zero_body, 0)

    kiters = _C // 16

    def _chunk_body(c, _):
        pltpu.sync_copy(l0w_hbm.at[:, pl.ds(c * _C, _C)], w0_buf)

        def _row_body(r, _):
            pltpu.sync_copy(white_hbm.at[base + r, pl.ds(c * _C, _C)], wf_buf)
            pltpu.sync_copy(black_hbm.at[base + r, pl.ds(c * _C, _C)], bf_buf)
            accs = tuple(acc_buf[pl.ds((r * 8 + j) * 16, 16)] for j in range(8))

            def _k_body(k, accs):
                wv = wf_buf[pl.ds(k * 16, 16)]
                bv = bf_buf[pl.ds(k * 16, 16)]
                out = list(accs)
                for m in range(4):
                    w0v = w0_buf[m, pl.ds(k * 16, 16)]
                    out[m] = out[m] + wv * w0v
                    out[4 + m] = out[4 + m] + bv * w0v
                return tuple(out)

            accs = lax.fori_loop(0, kiters, _k_body, accs)
            for j in range(8):
                acc_buf[pl.ds((r * 8 + j) * 16, 16)] = accs[j]
            return 0

        lax.fori_loop(0, rows_per_w, _row_body, 0)
        return 0

    lax.fori_loop(0, nchunks, _chunk_body, 0)

    lane = lax.iota(jnp.int32, 16)
    lane0 = lane == 0

    def _red_body(i, _):
        v = acc_buf[pl.ds(i * 16, 16)]
        for sh in (8, 4, 2, 1):
            red_buf[...] = v
            v = v + plsc.load_gather(red_buf, [lane ^ sh])
        plsc.store_scatter(sums_buf, [lane * 0 + i], v, mask=lane0)
        return 0

    lax.fori_loop(0, rows_per_w * 8, _red_body, 0)

    p0 = params_buf[pl.ds(0, 16)]
    p1 = params_buf[pl.ds(16, 16)]
    w1v = [params_buf[pl.ds(32 + 16 * q, 16)] for q in range(4)]
    l0b = [p0[m] for m in range(4)]
    l1b = [p0[8 + n] for n in range(8)]
    l2w = [p1[n] for n in range(8)]
    l2b = p1[8]
    l1w = [w1v[i // 16][i % 16] for i in range(64)]

    for g in range(rows_per_w // 16):
        t = turn_buf[pl.ds(g * 16, 16)]
        sums = []
        for j in range(8):
            col = plsc.load_gather(
                sums_buf,
                [lax.iota(jnp.int32, 16) * 8 + (g * 16 * 8 + j)])
            sums.append(col)
        w = [sums[m] + l0b[m] for m in range(4)]
        b = [sums[4 + m] + l0b[m] for m in range(4)]
        a = [t * w[m] + (1.0 - t) * b[m] for m in range(4)] + \
            [t * b[m] + (1.0 - t) * w[m] for m in range(4)]
        l1x = [jnp.clip(x, 0.0, 1.0) for x in a]
        out_v = jnp.zeros((16,), jnp.float32) + l2b
        for n in range(8):
            h = jnp.zeros((16,), jnp.float32) + l1b[n]
            for j in range(8):
                h = h + l1x[j] * l1w[n * 8 + j]
            l2x = jnp.clip(h, 0.0, 1.0)
            out_v = out_v + l2x * l2w[n]
        out_buf[pl.ds(g * 16, 16)] = out_v

    pltpu.sync_copy(out_buf, out_hbm.at[pl.ds(base, rows_per_w)])


def kernel(white_features, black_features, turn, score, result,
           l0_w, l0_b, l1_w, l1_b, l2_w, l2_b):
    del score, result
    B, F = white_features.shape
    M = l0_w.shape[0]
    N = l1_w.shape[0]
    K = l2_w.shape[0]

    info = plsc.get_sparse_core_info()
    nw = info.num_cores * info.num_subcores
    rows_per_w = B // nw
    nchunks = F // _C

    params = jnp.zeros((128,), jnp.float32)
    params = params.at[0:M].set(l0_b)
    params = params.at[8:8 + N].set(l1_b)
    params = params.at[16:16 + N].set(l2_w[0])
    params = params.at[24].set(l2_b[0])
    params = params.at[32:32 + N * 2 * M].set(l1_w.reshape(-1))

    mesh = plsc.VectorSubcoreMesh(core_axis_name="c", subcore_axis_name="s")
    body = functools.partial(_sc_body, B, F, rows_per_w, nchunks)

    out = pl.kernel(
        body,
        mesh=mesh,
        compiler_params=pltpu.CompilerParams(needs_layout_passes=False),
        out_type=jax.ShapeDtypeStruct((B,), jnp.float32),
        scratch_types=[
            pltpu.VMEM((_C,), jnp.float32),
            pltpu.VMEM((_C,), jnp.float32),
            pltpu.VMEM((M, _C), jnp.float32),
            pltpu.VMEM((rows_per_w * 8 * 16,), jnp.float32),
            pltpu.VMEM((rows_per_w * 8,), jnp.float32),
            pltpu.VMEM((rows_per_w,), jnp.float32),
            pltpu.VMEM((rows_per_w,), jnp.float32),
            pltpu.VMEM((128,), jnp.float32),
            pltpu.VMEM((16,), jnp.float32),
        ],
    )(white_features, black_features, turn.reshape(B), l0_w, params)
    return out.reshape(B, K)


# speedup vs baseline: 5.8643x; 5.8643x over previous
import functools

import jax
import jax.numpy as jnp
from jax import lax
from jax.experimental import pallas as pl
from jax.experimental.pallas import tpu as pltpu
from jax.experimental.pallas import tpu_sc as plsc

_C = 8192
_F_SC = 8192


def _sc_body(rows_per_w, f0, nchunks,
             white_hbm, black_hbm, l0w_hbm, out_hbm,
             wf_buf, bf_buf, w0_buf, acc_buf, sums_buf, red_buf):
    nc = lax.axis_index("c")
    ns = lax.axis_index("s")
    wid = ns * 2 + nc
    base = wid * rows_per_w

    zero = jnp.zeros((16,), jnp.float32)

    def _zero_body(i, _):
        acc_buf[pl.ds(i * 16, 16)] = zero
        return 0

    lax.fori_loop(0, rows_per_w * 8, _zero_body, 0)

    kiters = _C // 16

    def _chunk_body(c, _):
        pltpu.sync_copy(l0w_hbm.at[:, pl.ds(f0 + c * _C, _C)], w0_buf)

        def _row_body(r, _):
            pltpu.sync_copy(white_hbm.at[base + r, pl.ds(f0 + c * _C, _C)],
                            wf_buf)
            pltpu.sync_copy(black_hbm.at[base + r, pl.ds(f0 + c * _C, _C)],
                            bf_buf)
            accs = tuple(acc_buf[pl.ds((r * 8 + j) * 16, 16)] for j in range(8))

            def _k_body(k, accs):
                wv = wf_buf[pl.ds(k * 16, 16)]
                bv = bf_buf[pl.ds(k * 16, 16)]
                out = list(accs)
                for m in range(4):
                    w0v = w0_buf[m, pl.ds(k * 16, 16)]
                    out[m] = out[m] + wv * w0v
                    out[4 + m] = out[4 + m] + bv * w0v
                return tuple(out)

            accs = lax.fori_loop(0, kiters, _k_body, accs)
            for j in range(8):
                acc_buf[pl.ds((r * 8 + j) * 16, 16)] = accs[j]
            return 0

        lax.fori_loop(0, rows_per_w, _row_body, 0)
        return 0

    lax.fori_loop(0, nchunks, _chunk_body, 0)

    lane = lax.iota(jnp.int32, 16)
    lane0 = lane == 0

    def _red_body(i, _):
        v = acc_buf[pl.ds(i * 16, 16)]
        for sh in (8, 4, 2, 1):
            red_buf[...] = v
            v = v + plsc.load_gather(red_buf, [lane ^ sh])
        plsc.store_scatter(sums_buf, [lane * 0 + i], v, mask=lane0)
        return 0

    lax.fori_loop(0, rows_per_w * 8, _red_body, 0)

    pltpu.sync_copy(sums_buf, out_hbm.at[pl.ds(base * 8, rows_per_w * 8)])


def _tc_partial_body(wf_ref, bf_ref, l0w_ref, out_ref):
    w0 = l0w_ref[...]
    pw = jax.lax.dot_general(wf_ref[...], w0, (((1,), (1,)), ((), ())),
                             preferred_element_type=jnp.float32)
    pb = jax.lax.dot_general(bf_ref[...], w0, (((1,), (1,)), ((), ())),
                             preferred_element_type=jnp.float32)
    out_ref[...] = jnp.concatenate([pw, pb], axis=1)


def _combine_body(tcp_ref, scp_ref, turn_ref, l0b_ref, l1w_ref, l1b_ref,
                  l2w_ref, l2b_ref, out_ref):
    acc = tcp_ref[...] + scp_ref[...]
    m = acc.shape[1] // 2
    w = acc[:, :m] + l0b_ref[...]
    b = acc[:, m:] + l0b_ref[...]
    t = turn_ref[...]
    a = t * jnp.concatenate([w, b], axis=1) \
        + (1.0 - t) * jnp.concatenate([b, w], axis=1)
    l1_x = jnp.clip(a, 0.0, 1.0)
    h = jax.lax.dot_general(l1_x, l1w_ref[...], (((1,), (1,)), ((), ())),
                            preferred_element_type=jnp.float32) + l1b_ref[...]
    l2_x = jnp.clip(h, 0.0, 1.0)
    out_ref[...] = (jnp.sum(l2_x * l2w_ref[...], axis=1, keepdims=True)
                    + l2b_ref[0, 0])


def kernel(white_features, black_features, turn, score, result,
           l0_w, l0_b, l1_w, l1_b, l2_w, l2_b):
    del score, result
    B, F = white_features.shape
    M = l0_w.shape[0]
    N = l1_w.shape[0]
    K = l2_w.shape[0]

    f_sc = _F_SC if F > _F_SC else 0
    f_tc = F - f_sc

    info = plsc.get_sparse_core_info()
    nw = info.num_cores * info.num_subcores
    rows_per_w = B // nw
    mesh = plsc.VectorSubcoreMesh(core_axis_name="c", subcore_axis_name="s")
    sc_body = functools.partial(_sc_body, rows_per_w, f_tc, f_sc // _C)
    sc_flat = pl.kernel(
        sc_body,
        mesh=mesh,
        compiler_params=pltpu.CompilerParams(needs_layout_passes=False),
        out_type=jax.ShapeDtypeStruct((B * 2 * M,), jnp.float32),
        scratch_types=[
            pltpu.VMEM((_C,), jnp.float32),
            pltpu.VMEM((_C,), jnp.float32),
            pltpu.VMEM((M, _C), jnp.float32),
            pltpu.VMEM((rows_per_w * 8 * 16,), jnp.float32),
            pltpu.VMEM((rows_per_w * 8,), jnp.float32),
            pltpu.VMEM((16,), jnp.float32),
        ],
    )(white_features, black_features, l0_w)
    sc_partial = sc_flat.reshape(B, 2 * M)

    bt = 32 if B % 32 == 0 else B
    nb = B // bt
    tc_partial = pl.pallas_call(
        _tc_partial_body,
        grid=(nb,),
        in_specs=[
            pl.BlockSpec((bt, f_tc), lambda i: (i, 0)),
            pl.BlockSpec((bt, f_tc), lambda i: (i, 0)),
            pl.BlockSpec((M, f_tc), lambda i: (0, 0)),
        ],
        out_specs=pl.BlockSpec((bt, 2 * M), lambda i: (i, 0)),
        out_shape=jax.ShapeDtypeStruct((B, 2 * M), jnp.float32),
        compiler_params=pltpu.CompilerParams(
            dimension_semantics=("arbitrary",),
        ),
    )(white_features, black_features, l0_w)

    turn_b = jnp.broadcast_to(turn, (B, 2 * M))
    l0_b2 = l0_b.reshape(1, M)
    l1_b2 = l1_b.reshape(1, N)
    l2_b2 = l2_b.reshape(1, K)
    return pl.pallas_call(
        _combine_body,
        grid=(1,),
        in_specs=[
            pl.BlockSpec((B, 2 * M), lambda i: (0, 0)),
            pl.BlockSpec((B, 2 * M), lambda i: (0, 0)),
            pl.BlockSpec((B, 2 * M), lambda i: (0, 0)),
            pl.BlockSpec((1, M), lambda i: (0, 0)),
            pl.BlockSpec((N, 2 * M), lambda i: (0, 0)),
            pl.BlockSpec((1, N), lambda i: (0, 0)),
            pl.BlockSpec((K, N), lambda i: (0, 0)),
            pl.BlockSpec(memory_space=pltpu.SMEM),
        ],
        out_specs=pl.BlockSpec((B, K), lambda i: (0, 0)),
        out_shape=jax.ShapeDtypeStruct((B, K), jnp.float32),
    )(tc_partial, sc_partial, turn_b, l0_b2, l1_w, l1_b2, l2_w, l2_b2)
